# Initial kernel scaffold; baseline (speedup 1.0000x reference)
import jax
import jax.numpy as jnp
from jax.experimental import pallas as pl


def kernel(glimpse__feature, glimpse_member__local_pos, glimpse_member__log_mask, glimpse_member__glimpse_index, temperature, W, b):
    raise NotImplementedError("write your pallas kernel here")


# speedup vs baseline: 18.1094x; 18.1094x over previous
import dataclasses
import functools

import jax
import jax.numpy as jnp
from jax import lax
from jax.experimental import pallas as pl
from jax.experimental.pallas import tpu as pltpu
from jax.experimental.pallas import tpu_sc as plsc


def _sc_center_kernel(n_members, n_seg, seg_per_worker, tile, n_workers,
                      interpret=False):
  gp = seg_per_worker
  gpad = n_workers * gp
  t = tile
  assert n_members % t == 0 and t % 16 == 0 and gp % 16 == 0

  mesh = plsc.VectorSubcoreMesh(core_axis_name="c", subcore_axis_name="s",
                                num_cores=2, num_subcores=16)
  ns = 16
  cp = pltpu.CompilerParams()
  if "needs_layout_passes" in pltpu.CompilerParams.__dataclass_fields__:
    cp = dataclasses.replace(cp, needs_layout_passes=False)

  @functools.partial(
      pl.kernel,
      out_type=jax.ShapeDtypeStruct((2 * gpad,), jnp.float32),
      mesh=mesh,
      scratch_types=[
          pltpu.VMEM((2 * n_workers + 16,), jnp.int32),
          pltpu.VMEM((t + 32,), jnp.int32),
          pltpu.VMEM((t,), jnp.float32),
          pltpu.VMEM((2 * t,), jnp.float32),
          pltpu.VMEM((gp,), jnp.float32),
          pltpu.VMEM((gp,), jnp.float32),
          pltpu.VMEM((gp,), jnp.float32),
      ],
      compiler_params=cp,
      interpret=interpret,
  )
  def center_kernel(lm_hbm, pos_hbm, idx_hbm, bounds_hbm, out_hbm,
                    bnd_s, idx_buf, lm_buf, pos_buf, tab_e, tab_x, tab_y):
    w = lax.axis_index("c") * ns + lax.axis_index("s")
    pltpu.sync_copy(bounds_hbm, bnd_s.at[pl.ds(0, 2 * n_workers)])
    tlo = bnd_s[pl.ds(w, 16)][0]
    thi = bnd_s[pl.ds(n_workers + w, 16)][0]
    segbase = w * gp

    iota = lax.iota(jnp.int32, 16)
    zero16 = jnp.zeros((16,), jnp.float32)
    lane15 = iota == 15
    notl15 = iota != 15
    two_iota = iota * 2
    pad_val = jnp.full((16,), -(2 ** 30), jnp.int32)

    @pl.loop(0, gp, step=16)
    def _(j):
      tab_e[pl.ds(j, 16)] = zero16
      tab_x[pl.ds(j, 16)] = zero16
      tab_y[pl.ds(j, 16)] = zero16

    idx_buf[pl.ds(0, 16)] = pad_val
    idx_buf[pl.ds(16 + t, 16)] = pad_val

    @pl.loop(tlo, thi)
    def _(ti):
      m0 = ti * t
      pltpu.sync_copy(idx_hbm.at[pl.ds(m0, t)], idx_buf.at[pl.ds(16, t)])
      pltpu.sync_copy(lm_hbm.at[pl.ds(m0, t)], lm_buf)
      pltpu.sync_copy(pos_hbm.at[pl.ds(2 * m0, 2 * t)], pos_buf)

      @pl.loop(0, t, step=16)
      def _(v):
        idxv = idx_buf[pl.ds(16 + v, 16)]
        idxn = idx_buf[pl.ds(17 + v, 16)]
        lmv = lm_buf[pl.ds(v, 16)]
        e = jnp.exp(lmv)
        ix = two_iota + 2 * v
        gx = plsc.load_gather(pos_buf, [ix])
        gy = plsc.load_gather(pos_buf, [ix + 1])
        cse = plsc.cumsum(e)
        csx = plsc.cumsum(e * gx)
        csy = plsc.cumsum(e * gy)
        lidx = idxv - segbase
        lidxn = idxn - segbase
        inr = (lidx >= 0) & (lidx < gp)
        inrn = (lidxn >= 0) & (lidxn < gp)
        bnd = idxv != idxn
        em = (bnd | lane15) & inr
        em2 = bnd & notl15 & inrn
        plsc.addupdate_scatter(tab_e, [lidx], cse, mask=em)
        plsc.addupdate_scatter(tab_x, [lidx], csx, mask=em)
        plsc.addupdate_scatter(tab_y, [lidx], csy, mask=em)
        plsc.addupdate_scatter(tab_e, [lidxn], -cse, mask=em2)
        plsc.addupdate_scatter(tab_x, [lidxn], -csx, mask=em2)
        plsc.addupdate_scatter(tab_y, [lidxn], -csy, mask=em2)

    @pl.loop(0, gp, step=16)
    def _(j):
      ve = tab_e[pl.ds(j, 16)]
      nz = ve != 0.0
      inv = jnp.where(nz, 1.0, 0.0) / jnp.where(nz, ve, 1.0)
      tab_x[pl.ds(j, 16)] = tab_x[pl.ds(j, 16)] * inv
      tab_y[pl.ds(j, 16)] = tab_y[pl.ds(j, 16)] * inv

    pltpu.sync_copy(tab_x, out_hbm.at[pl.ds(segbase, gp)])
    pltpu.sync_copy(tab_y, out_hbm.at[pl.ds(gpad + segbase, gp)])

  return center_kernel


def _segment_centers(lm_flat, pos_flat, idx, n_seg, interpret=False,
                     tile=6400):
  n_members = idx.shape[0]
  n_workers = 32
  gp = (-(-n_seg // n_workers) + 15) // 16 * 16
  edges = (jnp.arange(n_workers + 1, dtype=jnp.int32) * gp).astype(idx.dtype)
  starts = jnp.searchsorted(idx, edges).astype(jnp.int32)
  tlo = starts[:-1] // tile
  thi = (starts[1:] + tile - 1) // tile
  bounds = jnp.concatenate([tlo, thi])
  out = _sc_center_kernel(n_members, n_seg, gp, tile, n_workers,
                          interpret=interpret)(lm_flat, pos_flat, idx, bounds)
  gpad = n_workers * gp
  return jnp.stack([out[:n_seg], out[gpad:gpad + n_seg]], axis=1)


def _tc_head_body(f_ref, w_ref, noise_ref, scal_ref, lp_ref, lz_ref):
  s = jnp.sum(f_ref[...] * w_ref[...], axis=1, keepdims=True)
  b = scal_ref[0, 0]
  inv_t = scal_ref[0, 1]
  lp = 8.8 * jnp.tanh(s + b)
  lp_ref[...] = lp
  relaxed = (lp + noise_ref[...]) * inv_t
  lz_ref[...] = jax.nn.log_sigmoid(relaxed)


def _dense_head(feature, w_row, noise, b, inv_t, block_rows):
  g, d = feature.shape
  assert g % block_rows == 0
  scal = jnp.stack([b.reshape(()).astype(jnp.float32),
                    jnp.asarray(inv_t, jnp.float32).reshape(())]).reshape(1, 2)
  grid = (g // block_rows,)
  lp, lz = pl.pallas_call(
      _tc_head_body,
      grid=grid,
      in_specs=[
          pl.BlockSpec((block_rows, d), lambda i: (i, 0)),
          pl.BlockSpec((1, d), lambda i: (0, 0)),
          pl.BlockSpec((block_rows, 1), lambda i: (i, 0)),
          pl.BlockSpec((1, 2), lambda i: (0, 0)),
      ],
      out_specs=[
          pl.BlockSpec((block_rows, 1), lambda i: (i, 0)),
          pl.BlockSpec((block_rows, 1), lambda i: (i, 0)),
      ],
      out_shape=[jax.ShapeDtypeStruct((g, 1), jnp.float32)] * 2,
  )(feature, w_row.reshape(1, d).astype(jnp.float32),
    noise.reshape(g, 1), scal)
  return lp[:, 0], lz[:, 0]


def kernel(glimpse__feature, glimpse_member__local_pos,
           glimpse_member__log_mask, glimpse_member__glimpse_index,
           temperature, W, b):
  g = glimpse__feature.shape[0]
  idx = glimpse_member__glimpse_index.astype(jnp.int32)
  lm_flat = glimpse_member__log_mask.reshape(-1)
  pos_flat = glimpse_member__local_pos.reshape(-1)

  member_center = _segment_centers(lm_flat, pos_flat, idx, g)

  u = jax.random.uniform(jax.random.key(42), (g,), minval=1e-6,
                         maxval=1.0 - 1e-6)
  noise = jnp.log(u) - jnp.log1p(-u)
  inv_t = 1.0 / jnp.asarray(temperature).astype(jnp.float32)
  logit_pres, log_z_pres = _dense_head(glimpse__feature, W, noise, b, inv_t,
                                       block_rows=4000)

  return (log_z_pres, logit_pres, member_center)


# speedup vs baseline: 18.2367x; 1.0070x over previous
import dataclasses
import functools

import jax
import jax.numpy as jnp
from jax import lax
from jax.experimental import pallas as pl
from jax.experimental.pallas import tpu as pltpu
from jax.experimental.pallas import tpu_sc as plsc


def _sc_center_kernel(n_members, n_seg, seg_per_worker, tile, n_workers,
                      interpret=False):
  gp = seg_per_worker
  gpad = n_workers * gp
  t = tile
  assert n_members % t == 0 and t % 16 == 0 and gp % 16 == 0

  mesh = plsc.VectorSubcoreMesh(core_axis_name="c", subcore_axis_name="s",
                                num_cores=2, num_subcores=16)
  ns = 16
  cp = pltpu.CompilerParams()
  if "needs_layout_passes" in pltpu.CompilerParams.__dataclass_fields__:
    cp = dataclasses.replace(cp, needs_layout_passes=False)

  @functools.partial(
      pl.kernel,
      out_type=jax.ShapeDtypeStruct((2 * gpad,), jnp.float32),
      mesh=mesh,
      scratch_types=[
          pltpu.VMEM((2 * n_workers + 16,), jnp.int32),
          pltpu.VMEM((t + 32,), jnp.int32),
          pltpu.VMEM((t,), jnp.float32),
          pltpu.VMEM((2 * t,), jnp.float32),
          pltpu.VMEM((t + 32,), jnp.int32),
          pltpu.VMEM((t,), jnp.float32),
          pltpu.VMEM((2 * t,), jnp.float32),
          pltpu.VMEM((gp,), jnp.float32),
          pltpu.VMEM((gp,), jnp.float32),
          pltpu.VMEM((gp,), jnp.float32),
          pltpu.SemaphoreType.DMA,
          pltpu.SemaphoreType.DMA,
          pltpu.SemaphoreType.DMA,
          pltpu.SemaphoreType.DMA,
          pltpu.SemaphoreType.DMA,
          pltpu.SemaphoreType.DMA,
      ],
      compiler_params=cp,
      interpret=interpret,
  )
  def center_kernel(lm_hbm, pos_hbm, idx_hbm, bounds_hbm, out_hbm,
                    bnd_s, idx_a, lm_a, pos_a, idx_b, lm_b, pos_b,
                    tab_e, tab_x, tab_y,
                    sia, sla, spa, sib, slb, spb):
    w = lax.axis_index("c") * ns + lax.axis_index("s")
    pltpu.sync_copy(bounds_hbm, bnd_s.at[pl.ds(0, 2 * n_workers)])
    tlo = bnd_s[pl.ds(w, 16)][0]
    thi = bnd_s[pl.ds(n_workers + w, 16)][0]
    segbase = w * gp

    iota = lax.iota(jnp.int32, 16)
    zero16 = jnp.zeros((16,), jnp.float32)
    lane15 = iota == 15
    notl15 = iota != 15
    pad_val = jnp.full((16,), -(2 ** 30), jnp.int32)

    @pl.loop(0, gp, step=16)
    def _(j):
      tab_e[pl.ds(j, 16)] = zero16
      tab_x[pl.ds(j, 16)] = zero16
      tab_y[pl.ds(j, 16)] = zero16

    for ib in (idx_a, idx_b):
      ib[pl.ds(0, 16)] = pad_val
      ib[pl.ds(16 + t, 16)] = pad_val

    def _copies(ti, idx_buf, lm_buf, pos_buf, si, sl, sp):
      m0 = ti * t
      return (
          (idx_hbm.at[pl.ds(m0, t)], idx_buf.at[pl.ds(16, t)], si),
          (lm_hbm.at[pl.ds(m0, t)], lm_buf, sl),
          (pos_hbm.at[pl.ds(2 * m0, 2 * t)], pos_buf, sp),
      )

    def issue(ti, bufs):
      for src, dst, sem in _copies(ti, *bufs):
        pltpu.async_copy(src, dst, sem)

    def wait(ti, bufs):
      for src, dst, sem in _copies(ti, *bufs):
        pltpu.make_async_copy(src, dst, sem).wait()

    def compute(bufs):
      idx_buf, lm_buf, pos_buf = bufs[:3]

      @plsc.parallel_loop(0, t, 16, unroll=2)
      def _(v):
        idxv = idx_buf[pl.ds(16 + v, 16)]
        idxn = idx_buf[pl.ds(17 + v, 16)]
        lmv = lm_buf[pl.ds(v, 16)]
        e = jnp.exp(lmv)
        ix = 2 * v + 2 * iota
        gx = plsc.load_gather(pos_buf, [ix])
        gy = plsc.load_gather(pos_buf, [ix + 1])
        cse = plsc.cumsum(e)
        csx = plsc.cumsum(e * gx)
        csy = plsc.cumsum(e * gy)
        lidx = idxv - segbase
        lidxn = idxn - segbase
        inr = (lidx >= 0) & (lidx < gp)
        inrn = (lidxn >= 0) & (lidxn < gp)
        bnd = idxv != idxn
        em = (bnd | lane15) & inr
        em2 = bnd & notl15 & inrn
        plsc.addupdate_scatter(tab_e, [lidx], cse, mask=em)
        plsc.addupdate_scatter(tab_x, [lidx], csx, mask=em)
        plsc.addupdate_scatter(tab_y, [lidx], csy, mask=em)
        plsc.addupdate_scatter(tab_e, [lidxn], -cse, mask=em2)
        plsc.addupdate_scatter(tab_x, [lidxn], -csx, mask=em2)
        plsc.addupdate_scatter(tab_y, [lidxn], -csy, mask=em2)

    bufs_a = (idx_a, lm_a, pos_a, sia, sla, spa)
    bufs_b = (idx_b, lm_b, pos_b, sib, slb, spb)

    @pl.when(tlo < thi)
    def _():
      issue(tlo, bufs_a)

    @pl.loop(tlo, thi, step=2)
    def _(ti):
      @pl.when(ti + 1 < thi)
      def _():
        issue(ti + 1, bufs_b)
      wait(ti, bufs_a)
      compute(bufs_a)

      @pl.when(ti + 2 < thi)
      def _():
        issue(ti + 2, bufs_a)

      @pl.when(ti + 1 < thi)
      def _():
        wait(ti + 1, bufs_b)
        compute(bufs_b)

    @pl.loop(0, gp, step=16)
    def _(j):
      ve = tab_e[pl.ds(j, 16)]
      nz = ve != 0.0
      inv = jnp.where(nz, 1.0, 0.0) / jnp.where(nz, ve, 1.0)
      tab_x[pl.ds(j, 16)] = tab_x[pl.ds(j, 16)] * inv
      tab_y[pl.ds(j, 16)] = tab_y[pl.ds(j, 16)] * inv

    pltpu.sync_copy(tab_x, out_hbm.at[pl.ds(segbase, gp)])
    pltpu.sync_copy(tab_y, out_hbm.at[pl.ds(gpad + segbase, gp)])

  return center_kernel


def _segment_centers(lm, pos, idx, n_seg, interpret=False, tile=6400):
  lm = lm.reshape(-1)
  pos = pos.reshape(-1)
  n_members = idx.shape[0]
  n_workers = 32
  gp = (-(-n_seg // n_workers) + 15) // 16 * 16
  edges = (jnp.arange(n_workers + 1, dtype=jnp.int32) * gp).astype(idx.dtype)
  starts = jnp.searchsorted(idx, edges).astype(jnp.int32)
  tlo = starts[:-1] // tile
  thi = (starts[1:] + tile - 1) // tile
  bounds = jnp.concatenate([tlo, thi])
  out = _sc_center_kernel(n_members, n_seg, gp, tile, n_workers,
                          interpret=interpret)(lm, pos, idx, bounds)
  gpad = n_workers * gp
  return jnp.stack([out[:n_seg], out[gpad:gpad + n_seg]], axis=1)


def _tc_head_body(f_ref, w_ref, noise_ref, scal_ref, lp_ref, lz_ref):
  s = jnp.sum(f_ref[...] * w_ref[...], axis=1, keepdims=True)
  b = scal_ref[0, 0]
  inv_t = scal_ref[0, 1]
  lp = 8.8 * jnp.tanh(s + b)
  lp_ref[...] = lp
  relaxed = (lp + noise_ref[...]) * inv_t
  lz_ref[...] = jax.nn.log_sigmoid(relaxed)


def _dense_head(feature, w_row, noise, b, inv_t, block_rows):
  g, d = feature.shape
  assert g % block_rows == 0
  scal = jnp.stack([b.reshape(()).astype(jnp.float32),
                    jnp.asarray(inv_t, jnp.float32).reshape(())]).reshape(1, 2)
  grid = (g // block_rows,)
  lp, lz = pl.pallas_call(
      _tc_head_body,
      grid=grid,
      in_specs=[
          pl.BlockSpec((block_rows, d), lambda i: (i, 0)),
          pl.BlockSpec((1, d), lambda i: (0, 0)),
          pl.BlockSpec((block_rows, 1), lambda i: (i, 0)),
          pl.BlockSpec((1, 2), lambda i: (0, 0)),
      ],
      out_specs=[
          pl.BlockSpec((block_rows, 1), lambda i: (i, 0)),
          pl.BlockSpec((block_rows, 1), lambda i: (i, 0)),
      ],
      out_shape=[jax.ShapeDtypeStruct((g, 1), jnp.float32)] * 2,
  )(feature, w_row.reshape(1, d).astype(jnp.float32),
    noise.reshape(g, 1), scal)
  return lp[:, 0], lz[:, 0]


def kernel(glimpse__feature, glimpse_member__local_pos,
           glimpse_member__log_mask, glimpse_member__glimpse_index,
           temperature, W, b):
  g = glimpse__feature.shape[0]
  idx = glimpse_member__glimpse_index.astype(jnp.int32)

  member_center = _segment_centers(glimpse_member__log_mask,
                                   glimpse_member__local_pos, idx, g)

  u = jax.random.uniform(jax.random.key(42), (g,), minval=1e-6,
                         maxval=1.0 - 1e-6)
  noise = jnp.log(u) - jnp.log1p(-u)
  inv_t = 1.0 / jnp.asarray(temperature).astype(jnp.float32)
  logit_pres, log_z_pres = _dense_head(glimpse__feature, W, noise, b, inv_t,
                                       block_rows=4000)

  return (log_z_pres, logit_pres, member_center)


# speedup vs baseline: 23.7376x; 1.3016x over previous
import dataclasses
import functools

import jax
import jax.numpy as jnp
from jax import lax
from jax.experimental import pallas as pl
from jax.experimental.pallas import tpu as pltpu
from jax.experimental.pallas import tpu_sc as plsc


def _tc_premul_body(lm_ref, pos_ref, e_ref, ex_ref, ey_ref):
  r = e_ref.shape[0]
  lmv = lm_ref[...][:, 0]
  e = jnp.exp(lmv)
  x = pos_ref[...][:, 0]
  y = pos_ref[...][:, 1]
  e_ref[...] = e.reshape(r, 128)
  ex_ref[...] = (e * x).reshape(r, 128)
  ey_ref[...] = (e * y).reshape(r, 128)


def _tc_premul(lm, pos, block_rows=10240):
  n = lm.shape[0]
  bt = block_rows
  assert n % bt == 0 and bt % 1024 == 0
  r = bt // 128
  outs = pl.pallas_call(
      _tc_premul_body,
      grid=(n // bt,),
      in_specs=[
          pl.BlockSpec((bt, 1), lambda i: (i, 0)),
          pl.BlockSpec((bt, 2), lambda i: (i, 0)),
      ],
      out_specs=[pl.BlockSpec((r, 128), lambda i: (i, 0))] * 3,
      out_shape=[jax.ShapeDtypeStruct((n // 128, 128), jnp.float32)] * 3,
  )(lm, pos)
  return tuple(o.reshape(n) for o in outs)


def _sc_center_kernel(n_members, n_seg, seg_per_worker, tile, n_workers,
                      interpret=False):
  gp = seg_per_worker
  gpad = n_workers * gp
  t = tile
  assert n_members % t == 0 and t % 16 == 0 and gp % 16 == 0

  mesh = plsc.VectorSubcoreMesh(core_axis_name="c", subcore_axis_name="s",
                                num_cores=2, num_subcores=16)
  ns = 16
  cp = pltpu.CompilerParams()
  if "needs_layout_passes" in pltpu.CompilerParams.__dataclass_fields__:
    cp = dataclasses.replace(cp, needs_layout_passes=False)

  @functools.partial(
      pl.kernel,
      out_type=jax.ShapeDtypeStruct((2 * gpad,), jnp.float32),
      mesh=mesh,
      scratch_types=[
          pltpu.VMEM((2 * n_workers + 16,), jnp.int32),
          pltpu.VMEM((t + 32,), jnp.int32),
          pltpu.VMEM((t,), jnp.float32),
          pltpu.VMEM((t,), jnp.float32),
          pltpu.VMEM((t,), jnp.float32),
          pltpu.VMEM((t + 32,), jnp.int32),
          pltpu.VMEM((t,), jnp.float32),
          pltpu.VMEM((t,), jnp.float32),
          pltpu.VMEM((t,), jnp.float32),
          pltpu.VMEM((gp,), jnp.float32),
          pltpu.VMEM((gp,), jnp.float32),
          pltpu.VMEM((gp,), jnp.float32),
          pltpu.SemaphoreType.DMA,
          pltpu.SemaphoreType.DMA,
          pltpu.SemaphoreType.DMA,
          pltpu.SemaphoreType.DMA,
          pltpu.SemaphoreType.DMA,
          pltpu.SemaphoreType.DMA,
          pltpu.SemaphoreType.DMA,
          pltpu.SemaphoreType.DMA,
      ],
      compiler_params=cp,
      interpret=interpret,
  )
  def center_kernel(e_hbm, ex_hbm, ey_hbm, idx_hbm, bounds_hbm, out_hbm,
                    bnd_s, idx_a, e_a, x_a, y_a, idx_b, e_b, x_b, y_b,
                    tab_e, tab_x, tab_y,
                    sia, sea, sxa, sya, sib, seb, sxb, syb):
    w = lax.axis_index("c") * ns + lax.axis_index("s")
    pltpu.sync_copy(bounds_hbm, bnd_s.at[pl.ds(0, 2 * n_workers)])
    tlo = bnd_s[pl.ds(w, 16)][0]
    thi = bnd_s[pl.ds(n_workers + w, 16)][0]
    segbase = w * gp

    iota = lax.iota(jnp.int32, 16)
    zero16 = jnp.zeros((16,), jnp.float32)
    lane15 = iota == 15
    notl15 = iota != 15
    pad_val = jnp.full((16,), -(2 ** 30), jnp.int32)

    @pl.loop(0, gp, step=16)
    def _(j):
      tab_e[pl.ds(j, 16)] = zero16
      tab_x[pl.ds(j, 16)] = zero16
      tab_y[pl.ds(j, 16)] = zero16

    for ib in (idx_a, idx_b):
      ib[pl.ds(0, 16)] = pad_val
      ib[pl.ds(16 + t, 16)] = pad_val

    def _copies(ti, idx_buf, e_buf, x_buf, y_buf, si, se, sx, sy):
      m0 = ti * t
      return (
          (idx_hbm.at[pl.ds(m0, t)], idx_buf.at[pl.ds(16, t)], si),
          (e_hbm.at[pl.ds(m0, t)], e_buf, se),
          (ex_hbm.at[pl.ds(m0, t)], x_buf, sx),
          (ey_hbm.at[pl.ds(m0, t)], y_buf, sy),
      )

    def issue(ti, bufs):
      for src, dst, sem in _copies(ti, *bufs):
        pltpu.async_copy(src, dst, sem)

    def wait(ti, bufs):
      for src, dst, sem in _copies(ti, *bufs):
        pltpu.make_async_copy(src, dst, sem).wait()

    def compute(bufs):
      idx_buf, e_buf, x_buf, y_buf = bufs[:4]

      @plsc.parallel_loop(0, t, 16, unroll=2)
      def _(v):
        idxv = idx_buf[pl.ds(16 + v, 16)]
        idxn = idx_buf[pl.ds(17 + v, 16)]
        e = e_buf[pl.ds(v, 16)]
        wx = x_buf[pl.ds(v, 16)]
        wy = y_buf[pl.ds(v, 16)]
        cse = plsc.cumsum(e)
        csx = plsc.cumsum(wx)
        csy = plsc.cumsum(wy)
        lidx = idxv - segbase
        lidxn = idxn - segbase
        inr = (lidx >= 0) & (lidx < gp)
        inrn = (lidxn >= 0) & (lidxn < gp)
        bnd = idxv != idxn
        em = (bnd | lane15) & inr
        em2 = bnd & notl15 & inrn
        plsc.addupdate_scatter(tab_e, [lidx], cse, mask=em)
        plsc.addupdate_scatter(tab_x, [lidx], csx, mask=em)
        plsc.addupdate_scatter(tab_y, [lidx], csy, mask=em)
        plsc.addupdate_scatter(tab_e, [lidxn], -cse, mask=em2)
        plsc.addupdate_scatter(tab_x, [lidxn], -csx, mask=em2)
        plsc.addupdate_scatter(tab_y, [lidxn], -csy, mask=em2)

    bufs_a = (idx_a, e_a, x_a, y_a, sia, sea, sxa, sya)
    bufs_b = (idx_b, e_b, x_b, y_b, sib, seb, sxb, syb)

    @pl.when(tlo < thi)
    def _():
      issue(tlo, bufs_a)

    @pl.loop(tlo, thi, step=2)
    def _(ti):
      @pl.when(ti + 1 < thi)
      def _():
        issue(ti + 1, bufs_b)
      wait(ti, bufs_a)
      compute(bufs_a)

      @pl.when(ti + 2 < thi)
      def _():
        issue(ti + 2, bufs_a)

      @pl.when(ti + 1 < thi)
      def _():
        wait(ti + 1, bufs_b)
        compute(bufs_b)

    @pl.loop(0, gp, step=16)
    def _(j):
      ve = tab_e[pl.ds(j, 16)]
      nz = ve != 0.0
      inv = jnp.where(nz, 1.0, 0.0) / jnp.where(nz, ve, 1.0)
      tab_x[pl.ds(j, 16)] = tab_x[pl.ds(j, 16)] * inv
      tab_y[pl.ds(j, 16)] = tab_y[pl.ds(j, 16)] * inv

    pltpu.sync_copy(tab_x, out_hbm.at[pl.ds(segbase, gp)])
    pltpu.sync_copy(tab_y, out_hbm.at[pl.ds(gpad + segbase, gp)])

  return center_kernel


def _segment_centers(e, ex, ey, idx, n_seg, interpret=False, tile=6400):
  n_members = idx.shape[0]
  n_workers = 32
  gp = (-(-n_seg // n_workers) + 15) // 16 * 16
  edges = (jnp.arange(n_workers + 1, dtype=jnp.int32) * gp).astype(idx.dtype)
  starts = jnp.searchsorted(idx, edges).astype(jnp.int32)
  tlo = starts[:-1] // tile
  thi = (starts[1:] + tile - 1) // tile
  bounds = jnp.concatenate([tlo, thi])
  out = _sc_center_kernel(n_members, n_seg, gp, tile, n_workers,
                          interpret=interpret)(e, ex, ey, idx, bounds)
  gpad = n_workers * gp
  return jnp.stack([out[:n_seg], out[gpad:gpad + n_seg]], axis=1)


def _tc_head_body(f_ref, w_ref, noise_ref, scal_ref, lp_ref, lz_ref):
  s = jnp.sum(f_ref[...] * w_ref[...], axis=1, keepdims=True)
  b = scal_ref[0, 0]
  inv_t = scal_ref[0, 1]
  lp = 8.8 * jnp.tanh(s + b)
  lp_ref[...] = lp
  relaxed = (lp + noise_ref[...]) * inv_t
  lz_ref[...] = jax.nn.log_sigmoid(relaxed)


def _dense_head(feature, w_row, noise, b, inv_t, block_rows):
  g, d = feature.shape
  assert g % block_rows == 0
  scal = jnp.stack([b.reshape(()).astype(jnp.float32),
                    jnp.asarray(inv_t, jnp.float32).reshape(())]).reshape(1, 2)
  grid = (g // block_rows,)
  lp, lz = pl.pallas_call(
      _tc_head_body,
      grid=grid,
      in_specs=[
          pl.BlockSpec((block_rows, d), lambda i: (i, 0)),
          pl.BlockSpec((1, d), lambda i: (0, 0)),
          pl.BlockSpec((block_rows, 1), lambda i: (i, 0)),
          pl.BlockSpec((1, 2), lambda i: (0, 0)),
      ],
      out_specs=[
          pl.BlockSpec((block_rows, 1), lambda i: (i, 0)),
          pl.BlockSpec((block_rows, 1), lambda i: (i, 0)),
      ],
      out_shape=[jax.ShapeDtypeStruct((g, 1), jnp.float32)] * 2,
  )(feature, w_row.reshape(1, d).astype(jnp.float32),
    noise.reshape(g, 1), scal)
  return lp[:, 0], lz[:, 0]


def kernel(glimpse__feature, glimpse_member__local_pos,
           glimpse_member__log_mask, glimpse_member__glimpse_index,
           temperature, W, b):
  g = glimpse__feature.shape[0]
  idx = glimpse_member__glimpse_index.astype(jnp.int32)

  e, ex, ey = _tc_premul(glimpse_member__log_mask, glimpse_member__local_pos)
  member_center = _segment_centers(e, ex, ey, idx, g)

  u = jax.random.uniform(jax.random.key(42), (g,), minval=1e-6,
                         maxval=1.0 - 1e-6)
  noise = jnp.log(u) - jnp.log1p(-u)
  inv_t = 1.0 / jnp.asarray(temperature).astype(jnp.float32)
  logit_pres, log_z_pres = _dense_head(glimpse__feature, W, noise, b, inv_t,
                                       block_rows=4000)

  return (log_z_pres, logit_pres, member_center)


# speedup vs baseline: 43.8490x; 1.8472x over previous
import dataclasses
import functools

import jax
import jax.numpy as jnp
from jax import lax
from jax.experimental import pallas as pl
from jax.experimental.pallas import tpu as pltpu
from jax.experimental.pallas import tpu_sc as plsc


def _tc_deint_body(pos_ref, px_ref, py_ref):
  r = px_ref.shape[0]
  pv = pos_ref[...]
  px_ref[...] = pv[:, 0].reshape(r, 128)
  py_ref[...] = pv[:, 1].reshape(r, 128)


def _tc_deint(pos, block_rows=10240):
  n = pos.shape[0]
  bt = block_rows
  assert n % bt == 0 and bt % 1024 == 0
  r = bt // 128
  outs = pl.pallas_call(
      _tc_deint_body,
      grid=(n // bt,),
      in_specs=[
          pl.BlockSpec((bt, 2), lambda i: (i, 0)),
      ],
      out_specs=[pl.BlockSpec((r, 128), lambda i: (i, 0))] * 2,
      out_shape=[jax.ShapeDtypeStruct((n // 128, 128), jnp.float32)] * 2,
  )(pos)
  return tuple(o.reshape(n) for o in outs)


def _sc_center_kernel(n_members, n_seg, seg_per_worker, tile, n_workers,
                      interpret=False):
  gp = seg_per_worker
  gpad = n_workers * gp
  t = tile
  assert n_members % t == 0 and t % 16 == 0 and gp % 16 == 0

  mesh = plsc.VectorSubcoreMesh(core_axis_name="c", subcore_axis_name="s",
                                num_cores=2, num_subcores=16)
  ns = 16
  cp = pltpu.CompilerParams()
  if "needs_layout_passes" in pltpu.CompilerParams.__dataclass_fields__:
    cp = dataclasses.replace(cp, needs_layout_passes=False)

  @functools.partial(
      pl.kernel,
      out_type=jax.ShapeDtypeStruct((2 * gpad,), jnp.float32),
      mesh=mesh,
      scratch_types=[
          pltpu.VMEM((2 * n_workers + 16,), jnp.int32),
          pltpu.VMEM((t + 32,), jnp.int32),
          pltpu.VMEM((t,), jnp.float32),
          pltpu.VMEM((t,), jnp.float32),
          pltpu.VMEM((t,), jnp.float32),
          pltpu.VMEM((t + 32,), jnp.int32),
          pltpu.VMEM((t,), jnp.float32),
          pltpu.VMEM((t,), jnp.float32),
          pltpu.VMEM((t,), jnp.float32),
          pltpu.VMEM((gp,), jnp.float32),
          pltpu.VMEM((gp,), jnp.float32),
          pltpu.VMEM((gp,), jnp.float32),
          pltpu.SemaphoreType.DMA,
          pltpu.SemaphoreType.DMA,
          pltpu.SemaphoreType.DMA,
          pltpu.SemaphoreType.DMA,
          pltpu.SemaphoreType.DMA,
          pltpu.SemaphoreType.DMA,
          pltpu.SemaphoreType.DMA,
          pltpu.SemaphoreType.DMA,
      ],
      compiler_params=cp,
      interpret=interpret,
  )
  def center_kernel(lm_hbm, px_hbm, py_hbm, idx_hbm, bounds_hbm, out_hbm,
                    bnd_s, idx_a, e_a, x_a, y_a, idx_b, e_b, x_b, y_b,
                    tab_e, tab_x, tab_y,
                    sia, sea, sxa, sya, sib, seb, sxb, syb):
    w = lax.axis_index("c") * ns + lax.axis_index("s")
    pltpu.sync_copy(bounds_hbm, bnd_s.at[pl.ds(0, 2 * n_workers)])
    tlo = bnd_s[pl.ds(w, 16)][0]
    thi = bnd_s[pl.ds(n_workers + w, 16)][0]
    segbase = w * gp

    iota = lax.iota(jnp.int32, 16)
    zero16 = jnp.zeros((16,), jnp.float32)
    lane15 = iota == 15
    notl15 = iota != 15
    pad_val = jnp.full((16,), -(2 ** 30), jnp.int32)

    @pl.loop(0, gp, step=16)
    def _(j):
      tab_e[pl.ds(j, 16)] = zero16
      tab_x[pl.ds(j, 16)] = zero16
      tab_y[pl.ds(j, 16)] = zero16

    for ib in (idx_a, idx_b):
      ib[pl.ds(0, 16)] = pad_val
      ib[pl.ds(16 + t, 16)] = pad_val

    def _copies(ti, idx_buf, e_buf, x_buf, y_buf, si, se, sx, sy):
      m0 = ti * t
      return (
          (idx_hbm.at[pl.ds(m0, t)], idx_buf.at[pl.ds(16, t)], si),
          (lm_hbm.at[pl.ds(m0, t)], e_buf, se),
          (px_hbm.at[pl.ds(m0, t)], x_buf, sx),
          (py_hbm.at[pl.ds(m0, t)], y_buf, sy),
      )

    def issue(ti, bufs):
      for src, dst, sem in _copies(ti, *bufs):
        pltpu.async_copy(src, dst, sem)

    def wait(ti, bufs):
      for src, dst, sem in _copies(ti, *bufs):
        pltpu.make_async_copy(src, dst, sem).wait()

    def compute(bufs):
      idx_buf, e_buf, x_buf, y_buf = bufs[:4]

      @plsc.parallel_loop(0, t, 16, unroll=2)
      def _(v):
        idxv = idx_buf[pl.ds(16 + v, 16)]
        idxn = idx_buf[pl.ds(17 + v, 16)]
        e = jnp.exp(e_buf[pl.ds(v, 16)])
        wx = e * x_buf[pl.ds(v, 16)]
        wy = e * y_buf[pl.ds(v, 16)]
        cse = plsc.cumsum(e)
        csx = plsc.cumsum(wx)
        csy = plsc.cumsum(wy)
        lidx = idxv - segbase
        lidxn = idxn - segbase
        inr = (lidx >= 0) & (lidx < gp)
        inrn = (lidxn >= 0) & (lidxn < gp)
        bnd = idxv != idxn
        em = (bnd | lane15) & inr
        em2 = bnd & notl15 & inrn
        plsc.addupdate_scatter(tab_e, [lidx], cse, mask=em)
        plsc.addupdate_scatter(tab_x, [lidx], csx, mask=em)
        plsc.addupdate_scatter(tab_y, [lidx], csy, mask=em)
        plsc.addupdate_scatter(tab_e, [lidxn], -cse, mask=em2)
        plsc.addupdate_scatter(tab_x, [lidxn], -csx, mask=em2)
        plsc.addupdate_scatter(tab_y, [lidxn], -csy, mask=em2)

    bufs_a = (idx_a, e_a, x_a, y_a, sia, sea, sxa, sya)
    bufs_b = (idx_b, e_b, x_b, y_b, sib, seb, sxb, syb)

    @pl.when(tlo < thi)
    def _():
      issue(tlo, bufs_a)

    @pl.loop(tlo, thi, step=2)
    def _(ti):
      @pl.when(ti + 1 < thi)
      def _():
        issue(ti + 1, bufs_b)
      wait(ti, bufs_a)
      compute(bufs_a)

      @pl.when(ti + 2 < thi)
      def _():
        issue(ti + 2, bufs_a)

      @pl.when(ti + 1 < thi)
      def _():
        wait(ti + 1, bufs_b)
        compute(bufs_b)

    @pl.loop(0, gp, step=16)
    def _(j):
      ve = tab_e[pl.ds(j, 16)]
      nz = ve != 0.0
      inv = jnp.where(nz, 1.0, 0.0) / jnp.where(nz, ve, 1.0)
      tab_x[pl.ds(j, 16)] = tab_x[pl.ds(j, 16)] * inv
      tab_y[pl.ds(j, 16)] = tab_y[pl.ds(j, 16)] * inv

    pltpu.sync_copy(tab_x, out_hbm.at[pl.ds(segbase, gp)])
    pltpu.sync_copy(tab_y, out_hbm.at[pl.ds(gpad + segbase, gp)])

  return center_kernel


def _segment_centers(lm, px, py, idx, n_seg, interpret=False, tile=6400):
  n_members = idx.shape[0]
  n_workers = 32
  gp = (-(-n_seg // n_workers) + 15) // 16 * 16
  edges = (jnp.arange(n_workers + 1, dtype=jnp.int32) * gp).astype(idx.dtype)
  starts = jnp.searchsorted(idx, edges).astype(jnp.int32)
  tlo = starts[:-1] // tile
  thi = (starts[1:] + tile - 1) // tile
  bounds = jnp.concatenate([tlo, thi])
  out = _sc_center_kernel(n_members, n_seg, gp, tile, n_workers,
                          interpret=interpret)(lm, px, py, idx, bounds)
  gpad = n_workers * gp
  return jnp.stack([out[:n_seg], out[gpad:gpad + n_seg]], axis=1)


def _tc_head_body(f_ref, w_ref, noise_ref, scal_ref, lp_ref, lz_ref):
  s = jnp.sum(f_ref[...] * w_ref[...], axis=1, keepdims=True)
  b = scal_ref[0, 0]
  inv_t = scal_ref[0, 1]
  lp = 8.8 * jnp.tanh(s + b)
  lp_ref[...] = lp
  relaxed = (lp + noise_ref[...]) * inv_t
  lz_ref[...] = jax.nn.log_sigmoid(relaxed)


def _dense_head(feature, w_row, noise, b, inv_t, block_rows):
  g, d = feature.shape
  assert g % block_rows == 0
  scal = jnp.stack([b.reshape(()).astype(jnp.float32),
                    jnp.asarray(inv_t, jnp.float32).reshape(())]).reshape(1, 2)
  grid = (g // block_rows,)
  lp, lz = pl.pallas_call(
      _tc_head_body,
      grid=grid,
      in_specs=[
          pl.BlockSpec((block_rows, d), lambda i: (i, 0)),
          pl.BlockSpec((1, d), lambda i: (0, 0)),
          pl.BlockSpec((block_rows, 1), lambda i: (i, 0)),
          pl.BlockSpec((1, 2), lambda i: (0, 0)),
      ],
      out_specs=[
          pl.BlockSpec((block_rows, 1), lambda i: (i, 0)),
          pl.BlockSpec((block_rows, 1), lambda i: (i, 0)),
      ],
      out_shape=[jax.ShapeDtypeStruct((g, 1), jnp.float32)] * 2,
  )(feature, w_row.reshape(1, d).astype(jnp.float32),
    noise.reshape(g, 1), scal)
  return lp[:, 0], lz[:, 0]


def kernel(glimpse__feature, glimpse_member__local_pos,
           glimpse_member__log_mask, glimpse_member__glimpse_index,
           temperature, W, b):
  g = glimpse__feature.shape[0]
  idx = glimpse_member__glimpse_index.astype(jnp.int32)

  lm_flat = glimpse_member__log_mask.reshape(-1)
  px, py = _tc_deint(glimpse_member__local_pos)
  member_center = _segment_centers(lm_flat, px, py, idx, g)

  u = jax.random.uniform(jax.random.key(42), (g,), minval=1e-6,
                         maxval=1.0 - 1e-6)
  noise = jnp.log(u) - jnp.log1p(-u)
  inv_t = 1.0 / jnp.asarray(temperature).astype(jnp.float32)
  logit_pres, log_z_pres = _dense_head(glimpse__feature, W, noise, b, inv_t,
                                       block_rows=4000)

  return (log_z_pres, logit_pres, member_center)
